# Initial kernel scaffold; baseline (speedup 1.0000x reference)
import jax
import jax.numpy as jnp
from jax.experimental import pallas as pl


def kernel(x, edge_index, batch, params):
    raise NotImplementedError("write your pallas kernel here")


# speedup vs baseline: 2.1699x; 2.1699x over previous
import functools

import jax
import jax.numpy as jnp
from jax import lax
from jax.experimental import pallas as pl
from jax.experimental.pallas import tpu as pltpu
from jax.experimental.pallas import tpu_sc as plsc

N = 10000
NP = 10240
E = 160000
EP = 163840
NBLK = EP // 128
BPW = NBLK // 32
ZR = NP // 16
G = 256
DUMMY = 10000
EPS = 1e-5
BLK = 1024
GRID = NP // BLK


def _sc_mesh():
    return plsc.VectorSubcoreMesh(core_axis_name="c", subcore_axis_name="s")


def _make_scatter(fc):
    def body(hp, rows, cols, zeros, out, ridx, cidx, gbuf, acc, sem):
        cid = lax.axis_index("c")
        sid = lax.axis_index("s")
        wid = sid * 2 + cid
        pltpu.sync_copy(zeros, acc.at[pl.ds(sid * ZR, ZR)])
        plsc.subcore_barrier()
        pltpu.sync_copy(rows.at[pl.ds(wid * BPW, BPW)], ridx)
        pltpu.sync_copy(cols.at[pl.ds(wid * BPW, BPW)], cidx)

        def step(j, carry):
            pltpu.async_copy(hp.at[ridx.at[j]], gbuf, sem).wait()
            pltpu.sync_copy(gbuf, acc.at[cidx.at[j]], add=True)
            return carry

        lax.fori_loop(0, BPW, step, 0)
        plsc.subcore_barrier()
        pltpu.sync_copy(acc.at[pl.ds(sid * ZR, ZR)],
                        out.at[cid, pl.ds(sid * ZR, ZR)])

    return pl.kernel(
        body,
        out_type=jax.ShapeDtypeStruct((2, NP, fc), jnp.float32),
        mesh=_sc_mesh(),
        scratch_types=[
            pltpu.VMEM((BPW, 128), jnp.int32),
            pltpu.VMEM((BPW, 128), jnp.int32),
            pltpu.VMEM((128, fc), jnp.float32),
            pltpu.VMEM_SHARED((NP, fc), jnp.float32),
            pltpu.SemaphoreType.DMA,
        ],
    )


def _make_deg():
    def body(rows, ones, zeros, out, ridx, obuf, acc):
        cid = lax.axis_index("c")
        sid = lax.axis_index("s")
        wid = sid * 2 + cid
        pltpu.sync_copy(zeros, acc.at[pl.ds(sid * ZR, ZR)])
        plsc.subcore_barrier()
        pltpu.sync_copy(rows.at[pl.ds(wid * BPW, BPW)], ridx)
        pltpu.sync_copy(ones, obuf)

        def step(j, carry):
            pltpu.sync_copy(obuf, acc.at[ridx.at[j]], add=True)
            return carry

        lax.fori_loop(0, BPW, step, 0)
        plsc.subcore_barrier()
        pltpu.sync_copy(acc.at[pl.ds(sid * ZR, ZR)],
                        out.at[cid, pl.ds(sid * ZR, ZR)])

    return pl.kernel(
        body,
        out_type=jax.ShapeDtypeStruct((2, NP, 128), jnp.float32),
        mesh=_sc_mesh(),
        scratch_types=[
            pltpu.VMEM((BPW, 128), jnp.int32),
            pltpu.VMEM((128, 128), jnp.float32),
            pltpu.VMEM_SHARED((NP, 128), jnp.float32),
        ],
    )


def _rows(blk):
    return pl.program_id(0) * blk + lax.broadcasted_iota(jnp.int32, (blk, 1), 0)


def _row_spec(f, blk=BLK):
    return pl.BlockSpec((blk, f), lambda i: (i, i * 0))


def _full_spec(shape):
    nd = len(shape)
    return pl.BlockSpec(shape, lambda i: (i * 0,) * nd)


def _dinv_body(d_ref, o_ref):
    o_ref[...] = lax.rsqrt(d_ref[0] + d_ref[1] + 1.0)


def _dinv(deg2):
    return pl.pallas_call(
        _dinv_body,
        grid=(1,),
        in_specs=[_full_spec((2, 80, 128))],
        out_specs=_full_spec((80, 128)),
        out_shape=jax.ShapeDtypeStruct((80, 128), jnp.float32),
    )(deg2.reshape(2, 80, 128)).reshape(NP, 1)


def _scale_body(x_ref, s_ref, o_ref):
    o_ref[...] = x_ref[...] * s_ref[...]


def _scale(x, s):
    f = x.shape[1]
    return pl.pallas_call(
        _scale_body,
        grid=(GRID,),
        in_specs=[_row_spec(f), _row_spec(1)],
        out_specs=_row_spec(f),
        out_shape=jax.ShapeDtypeStruct((NP, f), jnp.float32),
    )(x, s)


def _mm_body(x_ref, w_ref, b_ref, o_ref, *, act, mask):
    z = jnp.dot(x_ref[...], w_ref[...], preferred_element_type=jnp.float32)
    z = z + b_ref[...]
    if act:
        z = jnp.maximum(z, 0.0)
    if mask:
        z = jnp.where(_rows(BLK) < N, z, 0.0)
    o_ref[...] = z


def _mm(x, w, b, act, mask):
    k, n = w.shape
    return pl.pallas_call(
        functools.partial(_mm_body, act=act, mask=mask),
        grid=(GRID,),
        in_specs=[_row_spec(k), _full_spec((k, n)), _full_spec((1, n))],
        out_specs=_row_spec(n),
        out_shape=jax.ShapeDtypeStruct((NP, n), jnp.float32),
    )(x, w, b.reshape(1, n))


def _mm_scale_body(x_ref, w_ref, s_ref, o_ref):
    z = jnp.dot(x_ref[...], w_ref[...], preferred_element_type=jnp.float32)
    o_ref[...] = z * s_ref[...]


def _mm_scale(x, w, s):
    k, n = w.shape
    return pl.pallas_call(
        _mm_scale_body,
        grid=(GRID,),
        in_specs=[_row_spec(k), _full_spec((k, n)), _row_spec(1)],
        out_specs=_row_spec(n),
        out_shape=jax.ShapeDtypeStruct((NP, n), jnp.float32),
    )(x, w, s)


def _combine_body(*refs, nc, bias_relu):
    a_refs = refs[:nc]
    hp_ref, s_ref, b_ref, o_ref = refs[nc:]
    parts = [a[0] + a[1] for a in a_refs]
    acc = jnp.concatenate(parts, axis=1) if nc > 1 else parts[0]
    z = s_ref[...] * (acc + hp_ref[...])
    if bias_relu:
        z = jnp.maximum(z + b_ref[...], 0.0)
    o_ref[...] = jnp.where(_rows(BLK) < N, z, 0.0)


def _combine(accs, hp, s, b, bias_relu):
    f = hp.shape[1]
    nc = len(accs)
    cw = f // nc
    a_spec = pl.BlockSpec((2, BLK, cw), lambda i: (i * 0, i, i * 0))
    return pl.pallas_call(
        functools.partial(_combine_body, nc=nc, bias_relu=bias_relu),
        grid=(GRID,),
        in_specs=[a_spec] * nc + [_row_spec(f), _row_spec(1), _full_spec((1, f))],
        out_specs=_row_spec(f),
        out_shape=jax.ShapeDtypeStruct((NP, f), jnp.float32),
    )(*accs, hp, s, b.reshape(1, f))


def _bnsum_body(z_ref, o_ref):
    st = jnp.sum(z_ref[...], 0, keepdims=True)

    @pl.when(pl.program_id(0) == 0)
    def _():
        o_ref[...] = st

    @pl.when(pl.program_id(0) != 0)
    def _():
        o_ref[...] += st


def _bnsum(z):
    f = z.shape[1]
    return pl.pallas_call(
        _bnsum_body,
        grid=(GRID,),
        in_specs=[_row_spec(f)],
        out_specs=_full_spec((1, f)),
        out_shape=jax.ShapeDtypeStruct((1, f), jnp.float32),
    )(z)


def _bnvar_body(z_ref, sum_ref, o_ref):
    d = z_ref[...] - sum_ref[...] / N
    d = jnp.where(_rows(BLK) < N, d, 0.0)
    st = jnp.sum(d * d, 0, keepdims=True)

    @pl.when(pl.program_id(0) == 0)
    def _():
        o_ref[...] = st

    @pl.when(pl.program_id(0) != 0)
    def _():
        o_ref[...] += st


def _bnvar(z, s):
    f = z.shape[1]
    return pl.pallas_call(
        _bnvar_body,
        grid=(GRID,),
        in_specs=[_row_spec(f), _full_spec((1, f))],
        out_specs=_full_spec((1, f)),
        out_shape=jax.ShapeDtypeStruct((1, f), jnp.float32),
    )(z, s)


def _bnapply_body(z_ref, sum_ref, var_ref, g_ref, b_ref, s_ref, o_ref, *,
                  mul_s):
    m = sum_ref[...] / N
    var = var_ref[...] / N
    sc = g_ref[...] * lax.rsqrt(var + EPS)
    sh = b_ref[...] - m * sc
    z = z_ref[...] * sc + sh
    if mul_s:
        z = z * s_ref[...]
    o_ref[...] = jnp.where(_rows(BLK) < N, z, 0.0)


def _bnapply(z, g, b, s, mul_s):
    f = z.shape[1]
    zsum = _bnsum(z)
    zvar = _bnvar(z, zsum)
    return pl.pallas_call(
        functools.partial(_bnapply_body, mul_s=mul_s),
        grid=(GRID,),
        in_specs=[_row_spec(f), _full_spec((1, f)), _full_spec((1, f)),
                  _full_spec((1, f)), _full_spec((1, f)), _row_spec(1)],
        out_specs=_row_spec(f),
        out_shape=jax.ShapeDtypeStruct((NP, f), jnp.float32),
    )(z, zsum, zvar, g.reshape(1, f), b.reshape(1, f), s)


def _gmax_body(gate_ref, bat_ref, o_ref):
    g = gate_ref[:, 0:1]
    onehot = bat_ref[...] == lax.broadcasted_iota(jnp.int32, (BLK, G), 1)
    valid = _rows(BLK) < N
    mg = jnp.where(onehot & valid, g, -jnp.inf)
    mx = jnp.max(mg, 0, keepdims=True)

    @pl.when(pl.program_id(0) == 0)
    def _():
        o_ref[...] = mx

    @pl.when(pl.program_id(0) != 0)
    def _():
        o_ref[...] = jnp.maximum(o_ref[...], mx)


def _gmax(gate, bat_col):
    return pl.pallas_call(
        _gmax_body,
        grid=(GRID,),
        in_specs=[_row_spec(128), _row_spec(1)],
        out_specs=_full_spec((1, G)),
        out_shape=jax.ShapeDtypeStruct((1, G), jnp.float32),
    )(gate, bat_col)


def _exp_weights(gate_ref, bat_ref, gmax_ref):
    g = gate_ref[:, 0:1]
    onehot = (bat_ref[...] ==
              lax.broadcasted_iota(jnp.int32, (BLK, G), 1)).astype(jnp.float32)
    gm = jnp.where(jnp.isfinite(gmax_ref[...]), gmax_ref[...], 0.0)
    gmrow = jnp.sum(onehot * gm, 1, keepdims=True)
    valid = (_rows(BLK) < N).astype(jnp.float32)
    e = jnp.exp(g - gmrow) * valid
    return onehot, e


def _denom_body(gate_ref, bat_ref, gmax_ref, o_ref):
    onehot, e = _exp_weights(gate_ref, bat_ref, gmax_ref)
    contrib = jnp.sum(onehot * e, 0, keepdims=True)

    @pl.when(pl.program_id(0) == 0)
    def _():
        o_ref[...] = contrib

    @pl.when(pl.program_id(0) != 0)
    def _():
        o_ref[...] += contrib


def _denom(gate, bat_col, gmax):
    return pl.pallas_call(
        _denom_body,
        grid=(GRID,),
        in_specs=[_row_spec(128), _row_spec(1), _full_spec((1, G))],
        out_specs=_full_spec((1, G)),
        out_shape=jax.ShapeDtypeStruct((1, G), jnp.float32),
    )(gate, bat_col, gmax)


def _pool_body(h_ref, gate_ref, bat_ref, batr_ref, gmax_ref, den_ref, o_ref):
    onehot, e = _exp_weights(gate_ref, bat_ref, gmax_ref)
    d = jnp.sum(onehot * den_ref[...], 1, keepdims=True)
    alpha = e / jnp.where(d > 0, d, 1.0)
    w = alpha * h_ref[...]
    onehot_t = (lax.broadcasted_iota(jnp.int32, (G, BLK), 0) ==
                batr_ref[...]).astype(jnp.float32)
    contrib = jnp.dot(onehot_t, w, preferred_element_type=jnp.float32, precision=lax.Precision.HIGHEST)

    @pl.when(pl.program_id(0) == 0)
    def _():
        o_ref[...] = contrib

    @pl.when(pl.program_id(0) != 0)
    def _():
        o_ref[...] += contrib


def _pool(h5, gate, bat_col, bat_row, gmax, den):
    return pl.pallas_call(
        _pool_body,
        grid=(GRID,),
        in_specs=[_row_spec(1024), _row_spec(128), _row_spec(1),
                  pl.BlockSpec((1, BLK), lambda i: (i * 0, i)),
                  _full_spec((1, G)), _full_spec((1, G))],
        out_specs=_full_spec((G, 1024)),
        out_shape=jax.ShapeDtypeStruct((G, 1024), jnp.float32),
    )(h5, gate, bat_col, bat_row, gmax, den)


def _head_body(p_ref, w2_ref, b2_ref, w3_ref, b3_ref, w4_ref, b4_ref, o_ref):
    z = jnp.dot(p_ref[...], w2_ref[...], preferred_element_type=jnp.float32)
    z = jnp.maximum(z + b2_ref[...], 0.0)
    z = jnp.dot(z, w3_ref[...], preferred_element_type=jnp.float32)
    z = jnp.maximum(z + b3_ref[...], 0.0)
    z = jnp.dot(z, w4_ref[...], preferred_element_type=jnp.float32)
    o_ref[...] = z + b4_ref[...]


def _head(p, w2, b2, w3p, b3p, w4p, b4p):
    return pl.pallas_call(
        _head_body,
        grid=(1,),
        in_specs=[_full_spec((G, 1024)), _full_spec((1024, 128)),
                  _full_spec((1, 128)), _full_spec((128, 128)),
                  _full_spec((1, 128)), _full_spec((128, 128)),
                  _full_spec((1, 128))],
        out_specs=_full_spec((G, 128)),
        out_shape=jax.ShapeDtypeStruct((G, 128), jnp.float32),
    )(p, w2, b2, w3p, b3p, w4p, b4p)


def kernel(x, edge_index, batch, params):
    x = x.astype(jnp.float32)
    row = edge_index[0].astype(jnp.int32)
    col = edge_index[1].astype(jnp.int32)
    bat = batch.astype(jnp.int32)

    pad_e = EP - E
    rows_g = jnp.concatenate(
        [row, jnp.zeros((pad_e,), jnp.int32)]).reshape(NBLK, 128)
    rows_s = jnp.concatenate(
        [row, jnp.full((pad_e,), DUMMY, jnp.int32)]).reshape(NBLK, 128)
    cols_s = jnp.concatenate(
        [col, jnp.full((pad_e,), DUMMY, jnp.int32)]).reshape(NBLK, 128)
    xp = jnp.pad(x, ((0, NP - N), (0, 128 - x.shape[1])))
    bat_col = jnp.pad(bat, (0, NP - N)).reshape(NP, 1)
    bat_row = bat_col.reshape(1, NP)
    ones128 = jnp.ones((128, 128), jnp.float32)
    zeros128 = jnp.zeros((ZR, 128), jnp.float32)

    w1p = jnp.pad(params['W1'], ((0, 128 - 29), (0, 0)))
    gwp = jnp.pad(params['gate_W'], ((0, 0), (0, 127)))
    gbp = jnp.pad(params['gate_b'], (0, 127)).reshape(1, 128)
    w3p = jnp.pad(params['fc3_W'], ((0, 0), (0, 112)))
    b3p = jnp.pad(params['fc3_b'], (0, 112)).reshape(1, 128)
    w4p = jnp.pad(params['fc4_W'], ((0, 112), (0, 127)))
    b4p = jnp.pad(params['fc4_b'], (0, 127)).reshape(1, 128)

    deg2 = _make_deg()(rows_s, ones128, zeros128)[:, :, 0]
    dinv = _dinv(deg2)

    def propagate(hp):
        f = hp.shape[1]
        if f <= 128:
            return [_make_scatter(f)(hp, rows_g, cols_s, zeros128)]
        accs = []
        for c in range(f // 128):
            hpc = lax.slice_in_dim(hp, c * 128, (c + 1) * 128, axis=1)
            accs.append(_make_scatter(128)(hpc, rows_g, cols_s, zeros128))
        return accs

    h = xp
    ws = [w1p, params['W2'], params['W3'], params['W4'], params['W5']]
    for i in range(1, 6):
        mp = _mm_scale(h, ws[i - 1], dinv)
        z = _combine(propagate(mp), mp, dinv, params['b%d' % i], True)
        h = _bnapply(z, params['bn%d_g' % i], params['bn%d_b' % i], dinv,
                     False)

    gate = _mm(h, gwp, gbp.reshape(128), act=False, mask=True)
    gmax = _gmax(gate, bat_col)
    den = _denom(gate, bat_col, gmax)
    pooled = _pool(h, gate, bat_col, bat_row, gmax, den)

    out = _head(pooled, params['fc2_W'],
                params['fc2_b'].reshape(1, 128), w3p, b3p, w4p, b4p)
    return out[:, :1]


# speedup vs baseline: 2.2980x; 1.0590x over previous
import functools

import jax
import jax.numpy as jnp
from jax import lax
from jax.experimental import pallas as pl
from jax.experimental.pallas import tpu as pltpu
from jax.experimental.pallas import tpu_sc as plsc

N = 10000
NP = 10240
E = 160000
EP = 163840
NBLK = EP // 128
BPW = NBLK // 32
ZR = NP // 16
G = 256
DUMMY = 10000
EPS = 1e-5
BLK = 1024
GRID = NP // BLK


def _sc_mesh():
    return plsc.VectorSubcoreMesh(core_axis_name="c", subcore_axis_name="s")


def _make_scatter(fc):
    K = 2

    def body(hp, rows, cols, zeros, out, ridx, cidx, gbuf, acc, gsem):
        cid = lax.axis_index("c")
        sid = lax.axis_index("s")
        wid = sid * 2 + cid
        pltpu.sync_copy(zeros, acc.at[pl.ds(sid * ZR, ZR)])
        plsc.subcore_barrier()
        pltpu.sync_copy(rows.at[pl.ds(wid * BPW, BPW)], ridx)
        pltpu.sync_copy(cols.at[pl.ds(wid * BPW, BPW)], cidx)

        def group(g, carry):
            base = g * jnp.int32(K)
            gathers = []
            for t in range(K):
                gathers.append(pltpu.async_copy(
                    hp.at[ridx.at[base + jnp.int32(t)]],
                    gbuf.at[jnp.int32(t)], gsem))
            for t in range(K):
                gathers[t].wait()
                pltpu.sync_copy(gbuf.at[jnp.int32(t)],
                                acc.at[cidx.at[base + jnp.int32(t)]],
                                add=True)
            return carry

        lax.fori_loop(jnp.int32(0), jnp.int32(BPW // K), group, 0)
        plsc.subcore_barrier()
        pltpu.sync_copy(acc.at[pl.ds(sid * ZR, ZR)],
                        out.at[cid, pl.ds(sid * ZR, ZR)])

    return pl.kernel(
        body,
        out_type=jax.ShapeDtypeStruct((2, NP, fc), jnp.float32),
        mesh=_sc_mesh(),
        scratch_types=[
            pltpu.VMEM((BPW, 128), jnp.int32),
            pltpu.VMEM((BPW, 128), jnp.int32),
            pltpu.VMEM((K, 128, fc), jnp.float32),
            pltpu.VMEM_SHARED((NP, fc), jnp.float32),
            pltpu.SemaphoreType.DMA,
        ],
    )


def _make_deg():
    def body(rows, ones, zeros, out, ridx, obuf, acc):
        cid = lax.axis_index("c")
        sid = lax.axis_index("s")
        wid = sid * 2 + cid
        pltpu.sync_copy(zeros, acc.at[pl.ds(sid * ZR, ZR)])
        plsc.subcore_barrier()
        pltpu.sync_copy(rows.at[pl.ds(wid * BPW, BPW)], ridx)
        pltpu.sync_copy(ones, obuf)

        def step(j, carry):
            pltpu.sync_copy(obuf, acc.at[ridx.at[j]], add=True)
            return carry

        lax.fori_loop(0, BPW, step, 0)
        plsc.subcore_barrier()
        pltpu.sync_copy(acc.at[pl.ds(sid * ZR, ZR)],
                        out.at[cid, pl.ds(sid * ZR, ZR)])

    return pl.kernel(
        body,
        out_type=jax.ShapeDtypeStruct((2, NP, 128), jnp.float32),
        mesh=_sc_mesh(),
        scratch_types=[
            pltpu.VMEM((BPW, 128), jnp.int32),
            pltpu.VMEM((128, 128), jnp.float32),
            pltpu.VMEM_SHARED((NP, 128), jnp.float32),
        ],
    )


def _rows(blk):
    return pl.program_id(0) * blk + lax.broadcasted_iota(jnp.int32, (blk, 1), 0)


def _row_spec(f, blk=BLK):
    return pl.BlockSpec((blk, f), lambda i: (i, i * 0))


def _full_spec(shape):
    nd = len(shape)
    return pl.BlockSpec(shape, lambda i: (i * 0,) * nd)


def _dinv_body(d_ref, o_ref):
    o_ref[...] = lax.rsqrt(d_ref[0] + d_ref[1] + 1.0)


def _dinv(deg2):
    return pl.pallas_call(
        _dinv_body,
        grid=(1,),
        in_specs=[_full_spec((2, 80, 128))],
        out_specs=_full_spec((80, 128)),
        out_shape=jax.ShapeDtypeStruct((80, 128), jnp.float32),
    )(deg2.reshape(2, 80, 128)).reshape(NP, 1)


def _scale_body(x_ref, s_ref, o_ref):
    o_ref[...] = x_ref[...] * s_ref[...]


def _scale(x, s):
    f = x.shape[1]
    return pl.pallas_call(
        _scale_body,
        grid=(GRID,),
        in_specs=[_row_spec(f), _row_spec(1)],
        out_specs=_row_spec(f),
        out_shape=jax.ShapeDtypeStruct((NP, f), jnp.float32),
    )(x, s)


def _mm_body(x_ref, w_ref, b_ref, o_ref, *, act, mask):
    z = jnp.dot(x_ref[...], w_ref[...], preferred_element_type=jnp.float32)
    z = z + b_ref[...]
    if act:
        z = jnp.maximum(z, 0.0)
    if mask:
        z = jnp.where(_rows(BLK) < N, z, 0.0)
    o_ref[...] = z


def _mm(x, w, b, act, mask):
    k, n = w.shape
    return pl.pallas_call(
        functools.partial(_mm_body, act=act, mask=mask),
        grid=(GRID,),
        in_specs=[_row_spec(k), _full_spec((k, n)), _full_spec((1, n))],
        out_specs=_row_spec(n),
        out_shape=jax.ShapeDtypeStruct((NP, n), jnp.float32),
    )(x, w, b.reshape(1, n))


def _mm_scale_body(x_ref, w_ref, s_ref, o_ref):
    z = jnp.dot(x_ref[...], w_ref[...], preferred_element_type=jnp.float32)
    o_ref[...] = z * s_ref[...]


def _mm_scale(x, w, s):
    k, n = w.shape
    return pl.pallas_call(
        _mm_scale_body,
        grid=(GRID,),
        in_specs=[_row_spec(k), _full_spec((k, n)), _row_spec(1)],
        out_specs=_row_spec(n),
        out_shape=jax.ShapeDtypeStruct((NP, n), jnp.float32),
    )(x, w, s)


def _combine_body(*refs, nc, bias_relu):
    a_refs = refs[:nc]
    hp_ref, s_ref, b_ref, o_ref = refs[nc:]
    parts = [a[0] + a[1] for a in a_refs]
    acc = jnp.concatenate(parts, axis=1) if nc > 1 else parts[0]
    z = s_ref[...] * (acc + hp_ref[...])
    if bias_relu:
        z = jnp.maximum(z + b_ref[...], 0.0)
    o_ref[...] = jnp.where(_rows(BLK) < N, z, 0.0)


def _combine(accs, hp, s, b, bias_relu):
    f = hp.shape[1]
    nc = len(accs)
    cw = f // nc
    a_spec = pl.BlockSpec((2, BLK, cw), lambda i: (i * 0, i, i * 0))
    return pl.pallas_call(
        functools.partial(_combine_body, nc=nc, bias_relu=bias_relu),
        grid=(GRID,),
        in_specs=[a_spec] * nc + [_row_spec(f), _row_spec(1), _full_spec((1, f))],
        out_specs=_row_spec(f),
        out_shape=jax.ShapeDtypeStruct((NP, f), jnp.float32),
    )(*accs, hp, s, b.reshape(1, f))


def _bnsum_body(z_ref, o_ref):
    st = jnp.sum(z_ref[...], 0, keepdims=True)

    @pl.when(pl.program_id(0) == 0)
    def _():
        o_ref[...] = st

    @pl.when(pl.program_id(0) != 0)
    def _():
        o_ref[...] += st


def _bnsum(z):
    f = z.shape[1]
    return pl.pallas_call(
        _bnsum_body,
        grid=(GRID,),
        in_specs=[_row_spec(f)],
        out_specs=_full_spec((1, f)),
        out_shape=jax.ShapeDtypeStruct((1, f), jnp.float32),
    )(z)


def _bnvar_body(z_ref, sum_ref, o_ref):
    d = z_ref[...] - sum_ref[...] / N
    d = jnp.where(_rows(BLK) < N, d, 0.0)
    st = jnp.sum(d * d, 0, keepdims=True)

    @pl.when(pl.program_id(0) == 0)
    def _():
        o_ref[...] = st

    @pl.when(pl.program_id(0) != 0)
    def _():
        o_ref[...] += st


def _bnvar(z, s):
    f = z.shape[1]
    return pl.pallas_call(
        _bnvar_body,
        grid=(GRID,),
        in_specs=[_row_spec(f), _full_spec((1, f))],
        out_specs=_full_spec((1, f)),
        out_shape=jax.ShapeDtypeStruct((1, f), jnp.float32),
    )(z, s)


def _bnapply_body(z_ref, sum_ref, var_ref, g_ref, b_ref, s_ref, o_ref, *,
                  mul_s):
    m = sum_ref[...] / N
    var = var_ref[...] / N
    sc = g_ref[...] * lax.rsqrt(var + EPS)
    sh = b_ref[...] - m * sc
    z = z_ref[...] * sc + sh
    if mul_s:
        z = z * s_ref[...]
    o_ref[...] = jnp.where(_rows(BLK) < N, z, 0.0)


def _bnapply(z, g, b, s, mul_s):
    f = z.shape[1]
    zsum = _bnsum(z)
    zvar = _bnvar(z, zsum)
    return pl.pallas_call(
        functools.partial(_bnapply_body, mul_s=mul_s),
        grid=(GRID,),
        in_specs=[_row_spec(f), _full_spec((1, f)), _full_spec((1, f)),
                  _full_spec((1, f)), _full_spec((1, f)), _row_spec(1)],
        out_specs=_row_spec(f),
        out_shape=jax.ShapeDtypeStruct((NP, f), jnp.float32),
    )(z, zsum, zvar, g.reshape(1, f), b.reshape(1, f), s)


def _gmax_body(gate_ref, bat_ref, o_ref):
    g = gate_ref[:, 0:1]
    onehot = bat_ref[...] == lax.broadcasted_iota(jnp.int32, (BLK, G), 1)
    valid = _rows(BLK) < N
    mg = jnp.where(onehot & valid, g, -jnp.inf)
    mx = jnp.max(mg, 0, keepdims=True)

    @pl.when(pl.program_id(0) == 0)
    def _():
        o_ref[...] = mx

    @pl.when(pl.program_id(0) != 0)
    def _():
        o_ref[...] = jnp.maximum(o_ref[...], mx)


def _gmax(gate, bat_col):
    return pl.pallas_call(
        _gmax_body,
        grid=(GRID,),
        in_specs=[_row_spec(128), _row_spec(1)],
        out_specs=_full_spec((1, G)),
        out_shape=jax.ShapeDtypeStruct((1, G), jnp.float32),
    )(gate, bat_col)


def _exp_weights(gate_ref, bat_ref, gmax_ref):
    g = gate_ref[:, 0:1]
    onehot = (bat_ref[...] ==
              lax.broadcasted_iota(jnp.int32, (BLK, G), 1)).astype(jnp.float32)
    gm = jnp.where(jnp.isfinite(gmax_ref[...]), gmax_ref[...], 0.0)
    gmrow = jnp.sum(onehot * gm, 1, keepdims=True)
    valid = (_rows(BLK) < N).astype(jnp.float32)
    e = jnp.exp(g - gmrow) * valid
    return onehot, e


def _denom_body(gate_ref, bat_ref, gmax_ref, o_ref):
    onehot, e = _exp_weights(gate_ref, bat_ref, gmax_ref)
    contrib = jnp.sum(onehot * e, 0, keepdims=True)

    @pl.when(pl.program_id(0) == 0)
    def _():
        o_ref[...] = contrib

    @pl.when(pl.program_id(0) != 0)
    def _():
        o_ref[...] += contrib


def _denom(gate, bat_col, gmax):
    return pl.pallas_call(
        _denom_body,
        grid=(GRID,),
        in_specs=[_row_spec(128), _row_spec(1), _full_spec((1, G))],
        out_specs=_full_spec((1, G)),
        out_shape=jax.ShapeDtypeStruct((1, G), jnp.float32),
    )(gate, bat_col, gmax)


def _pool_body(h_ref, gate_ref, bat_ref, batr_ref, gmax_ref, den_ref, o_ref):
    onehot, e = _exp_weights(gate_ref, bat_ref, gmax_ref)
    d = jnp.sum(onehot * den_ref[...], 1, keepdims=True)
    alpha = e / jnp.where(d > 0, d, 1.0)
    w = alpha * h_ref[...]
    onehot_t = (lax.broadcasted_iota(jnp.int32, (G, BLK), 0) ==
                batr_ref[...]).astype(jnp.float32)
    contrib = jnp.dot(onehot_t, w, preferred_element_type=jnp.float32, precision=lax.Precision.HIGHEST)

    @pl.when(pl.program_id(0) == 0)
    def _():
        o_ref[...] = contrib

    @pl.when(pl.program_id(0) != 0)
    def _():
        o_ref[...] += contrib


def _pool(h5, gate, bat_col, bat_row, gmax, den):
    return pl.pallas_call(
        _pool_body,
        grid=(GRID,),
        in_specs=[_row_spec(1024), _row_spec(128), _row_spec(1),
                  pl.BlockSpec((1, BLK), lambda i: (i * 0, i)),
                  _full_spec((1, G)), _full_spec((1, G))],
        out_specs=_full_spec((G, 1024)),
        out_shape=jax.ShapeDtypeStruct((G, 1024), jnp.float32),
    )(h5, gate, bat_col, bat_row, gmax, den)


def _head_body(p_ref, w2_ref, b2_ref, w3_ref, b3_ref, w4_ref, b4_ref, o_ref):
    z = jnp.dot(p_ref[...], w2_ref[...], preferred_element_type=jnp.float32)
    z = jnp.maximum(z + b2_ref[...], 0.0)
    z = jnp.dot(z, w3_ref[...], preferred_element_type=jnp.float32)
    z = jnp.maximum(z + b3_ref[...], 0.0)
    z = jnp.dot(z, w4_ref[...], preferred_element_type=jnp.float32)
    o_ref[...] = z + b4_ref[...]


def _head(p, w2, b2, w3p, b3p, w4p, b4p):
    return pl.pallas_call(
        _head_body,
        grid=(1,),
        in_specs=[_full_spec((G, 1024)), _full_spec((1024, 128)),
                  _full_spec((1, 128)), _full_spec((128, 128)),
                  _full_spec((1, 128)), _full_spec((128, 128)),
                  _full_spec((1, 128))],
        out_specs=_full_spec((G, 128)),
        out_shape=jax.ShapeDtypeStruct((G, 128), jnp.float32),
    )(p, w2, b2, w3p, b3p, w4p, b4p)


def kernel(x, edge_index, batch, params):
    x = x.astype(jnp.float32)
    row = edge_index[0].astype(jnp.int32)
    col = edge_index[1].astype(jnp.int32)
    bat = batch.astype(jnp.int32)

    pad_e = EP - E
    rows_g = jnp.concatenate(
        [row, jnp.zeros((pad_e,), jnp.int32)]).reshape(NBLK, 128)
    rows_s = jnp.concatenate(
        [row, jnp.full((pad_e,), DUMMY, jnp.int32)]).reshape(NBLK, 128)
    cols_s = jnp.concatenate(
        [col, jnp.full((pad_e,), DUMMY, jnp.int32)]).reshape(NBLK, 128)
    xp = jnp.pad(x, ((0, NP - N), (0, 128 - x.shape[1])))
    bat_col = jnp.pad(bat, (0, NP - N)).reshape(NP, 1)
    bat_row = bat_col.reshape(1, NP)
    ones128 = jnp.ones((128, 128), jnp.float32)
    zeros128 = jnp.zeros((ZR, 128), jnp.float32)

    w1p = jnp.pad(params['W1'], ((0, 128 - 29), (0, 0)))
    gwp = jnp.pad(params['gate_W'], ((0, 0), (0, 127)))
    gbp = jnp.pad(params['gate_b'], (0, 127)).reshape(1, 128)
    w3p = jnp.pad(params['fc3_W'], ((0, 0), (0, 112)))
    b3p = jnp.pad(params['fc3_b'], (0, 112)).reshape(1, 128)
    w4p = jnp.pad(params['fc4_W'], ((0, 112), (0, 127)))
    b4p = jnp.pad(params['fc4_b'], (0, 127)).reshape(1, 128)

    deg2 = _make_deg()(rows_s, ones128, zeros128)[:, :, 0]
    dinv = _dinv(deg2)

    def propagate(hp):
        f = hp.shape[1]
        if f <= 128:
            return [_make_scatter(f)(hp, rows_g, cols_s, zeros128)]
        accs = []
        for c in range(f // 128):
            hpc = lax.slice_in_dim(hp, c * 128, (c + 1) * 128, axis=1)
            accs.append(_make_scatter(128)(hpc, rows_g, cols_s, zeros128))
        return accs

    h = xp
    ws = [w1p, params['W2'], params['W3'], params['W4'], params['W5']]
    for i in range(1, 6):
        mp = _mm_scale(h, ws[i - 1], dinv)
        z = _combine(propagate(mp), mp, dinv, params['b%d' % i], True)
        h = _bnapply(z, params['bn%d_g' % i], params['bn%d_b' % i], dinv,
                     False)

    gate = _mm(h, gwp, gbp.reshape(128), act=False, mask=True)
    gmax = _gmax(gate, bat_col)
    den = _denom(gate, bat_col, gmax)
    pooled = _pool(h, gate, bat_col, bat_row, gmax, den)

    out = _head(pooled, params['fc2_W'],
                params['fc2_b'].reshape(1, 128), w3p, b3p, w4p, b4p)
    return out[:, :1]


# speedup vs baseline: 2.4269x; 1.0561x over previous
import functools

import jax
import jax.numpy as jnp
from jax import lax
from jax.experimental import pallas as pl
from jax.experimental.pallas import tpu as pltpu
from jax.experimental.pallas import tpu_sc as plsc

N = 10000
NP = 10240
E = 160000
EP = 163840
NBLK = EP // 128
BPW = NBLK // 32
ZR = NP // 16
G = 256
DUMMY = 10000
EPS = 1e-5
BLK = 1024
GRID = NP // BLK


def _sc_mesh():
    return plsc.VectorSubcoreMesh(core_axis_name="c", subcore_axis_name="s")


def _make_scatter(nc):
    K = 2

    def body(*refs):
        hps = refs[:nc]
        rows, cols, zeros, out = refs[nc:nc + 4]
        ridx, cidx, gbuf, acc, gsem = refs[nc + 4:]
        cid = lax.axis_index("c")
        sid = lax.axis_index("s")
        wid = sid * 2 + cid
        pltpu.sync_copy(rows.at[pl.ds(wid * BPW, BPW)], ridx)
        pltpu.sync_copy(cols.at[pl.ds(wid * BPW, BPW)], cidx)

        for c in range(nc):
            pltpu.sync_copy(zeros, acc.at[pl.ds(sid * ZR, ZR)])
            plsc.subcore_barrier()

            def group(g, carry, hp=hps[c]):
                base = g * jnp.int32(K)
                gathers = []
                for t in range(K):
                    gathers.append(pltpu.async_copy(
                        hp.at[ridx.at[base + jnp.int32(t)]],
                        gbuf.at[jnp.int32(t)], gsem))
                for t in range(K):
                    gathers[t].wait()
                    pltpu.sync_copy(gbuf.at[jnp.int32(t)],
                                    acc.at[cidx.at[base + jnp.int32(t)]],
                                    add=True)
                return carry

            lax.fori_loop(jnp.int32(0), jnp.int32(BPW // K), group, 0)
            plsc.subcore_barrier()
            pltpu.sync_copy(acc.at[pl.ds(sid * ZR, ZR)],
                            out.at[jnp.int32(c), cid, pl.ds(sid * ZR, ZR)])

    return pl.kernel(
        body,
        out_type=jax.ShapeDtypeStruct((nc, 2, NP, 128), jnp.float32),
        mesh=_sc_mesh(),
        scratch_types=[
            pltpu.VMEM((BPW, 128), jnp.int32),
            pltpu.VMEM((BPW, 128), jnp.int32),
            pltpu.VMEM((K, 128, 128), jnp.float32),
            pltpu.VMEM_SHARED((NP, 128), jnp.float32),
            pltpu.SemaphoreType.DMA,
        ],
    )


def _make_deg():
    def body(rows, ones, zeros, out, ridx, obuf, acc):
        cid = lax.axis_index("c")
        sid = lax.axis_index("s")
        wid = sid * 2 + cid
        pltpu.sync_copy(zeros, acc.at[pl.ds(sid * ZR, ZR)])
        plsc.subcore_barrier()
        pltpu.sync_copy(rows.at[pl.ds(wid * BPW, BPW)], ridx)
        pltpu.sync_copy(ones, obuf)

        def step(j, carry):
            pltpu.sync_copy(obuf, acc.at[ridx.at[j]], add=True)
            return carry

        lax.fori_loop(0, BPW, step, 0)
        plsc.subcore_barrier()
        pltpu.sync_copy(acc.at[pl.ds(sid * ZR, ZR)],
                        out.at[cid, pl.ds(sid * ZR, ZR)])

    return pl.kernel(
        body,
        out_type=jax.ShapeDtypeStruct((2, NP, 128), jnp.float32),
        mesh=_sc_mesh(),
        scratch_types=[
            pltpu.VMEM((BPW, 128), jnp.int32),
            pltpu.VMEM((128, 128), jnp.float32),
            pltpu.VMEM_SHARED((NP, 128), jnp.float32),
        ],
    )


def _rows(blk):
    return pl.program_id(0) * blk + lax.broadcasted_iota(jnp.int32, (blk, 1), 0)


def _row_spec(f, blk=BLK):
    return pl.BlockSpec((blk, f), lambda i: (i, i * 0))


def _full_spec(shape):
    nd = len(shape)
    return pl.BlockSpec(shape, lambda i: (i * 0,) * nd)


def _dinv_body(d_ref, o_ref):
    o_ref[...] = lax.rsqrt(d_ref[0] + d_ref[1] + 1.0)


def _dinv(deg2):
    return pl.pallas_call(
        _dinv_body,
        grid=(1,),
        in_specs=[_full_spec((2, 80, 128))],
        out_specs=_full_spec((80, 128)),
        out_shape=jax.ShapeDtypeStruct((80, 128), jnp.float32),
    )(deg2.reshape(2, 80, 128)).reshape(NP, 1)


def _scale_body(x_ref, s_ref, o_ref):
    o_ref[...] = x_ref[...] * s_ref[...]


def _scale(x, s):
    f = x.shape[1]
    return pl.pallas_call(
        _scale_body,
        grid=(GRID,),
        in_specs=[_row_spec(f), _row_spec(1)],
        out_specs=_row_spec(f),
        out_shape=jax.ShapeDtypeStruct((NP, f), jnp.float32),
    )(x, s)


def _mm_body(x_ref, w_ref, b_ref, o_ref, *, act, mask):
    z = jnp.dot(x_ref[...], w_ref[...], preferred_element_type=jnp.float32)
    z = z + b_ref[...]
    if act:
        z = jnp.maximum(z, 0.0)
    if mask:
        z = jnp.where(_rows(BLK) < N, z, 0.0)
    o_ref[...] = z


def _mm(x, w, b, act, mask):
    k, n = w.shape
    return pl.pallas_call(
        functools.partial(_mm_body, act=act, mask=mask),
        grid=(GRID,),
        in_specs=[_row_spec(k), _full_spec((k, n)), _full_spec((1, n))],
        out_specs=_row_spec(n),
        out_shape=jax.ShapeDtypeStruct((NP, n), jnp.float32),
    )(x, w, b.reshape(1, n))


def _mm_scale_body(x_ref, w_ref, s_ref, o_ref):
    z = jnp.dot(x_ref[...], w_ref[...], preferred_element_type=jnp.float32)
    o_ref[...] = z * s_ref[...]


def _mm_scale(x, w, s):
    k, n = w.shape
    return pl.pallas_call(
        _mm_scale_body,
        grid=(GRID,),
        in_specs=[_row_spec(k), _full_spec((k, n)), _row_spec(1)],
        out_specs=_row_spec(n),
        out_shape=jax.ShapeDtypeStruct((NP, n), jnp.float32),
    )(x, w, s)


def _combine_body(a_ref, hp_ref, s_ref, b_ref, o_ref, *, nc, bias_relu):
    parts = [a_ref[c, 0] + a_ref[c, 1] for c in range(nc)]
    acc = jnp.concatenate(parts, axis=1) if nc > 1 else parts[0]
    z = s_ref[...] * (acc + hp_ref[...])
    if bias_relu:
        z = jnp.maximum(z + b_ref[...], 0.0)
    o_ref[...] = jnp.where(_rows(BLK) < N, z, 0.0)


def _combine(acc_all, hp, s, b, bias_relu):
    f = hp.shape[1]
    nc = f // 128
    a_spec = pl.BlockSpec((nc, 2, BLK, 128),
                          lambda i: (i * 0, i * 0, i, i * 0))
    return pl.pallas_call(
        functools.partial(_combine_body, nc=nc, bias_relu=bias_relu),
        grid=(GRID,),
        in_specs=[a_spec, _row_spec(f), _row_spec(1), _full_spec((1, f))],
        out_specs=_row_spec(f),
        out_shape=jax.ShapeDtypeStruct((NP, f), jnp.float32),
    )(acc_all, hp, s, b.reshape(1, f))


def _bnsum_body(z_ref, o_ref):
    st = jnp.sum(z_ref[...], 0, keepdims=True)

    @pl.when(pl.program_id(0) == 0)
    def _():
        o_ref[...] = st

    @pl.when(pl.program_id(0) != 0)
    def _():
        o_ref[...] += st


def _bnsum(z):
    f = z.shape[1]
    return pl.pallas_call(
        _bnsum_body,
        grid=(GRID,),
        in_specs=[_row_spec(f)],
        out_specs=_full_spec((1, f)),
        out_shape=jax.ShapeDtypeStruct((1, f), jnp.float32),
    )(z)


def _bnvar_body(z_ref, sum_ref, o_ref):
    d = z_ref[...] - sum_ref[...] / N
    d = jnp.where(_rows(BLK) < N, d, 0.0)
    st = jnp.sum(d * d, 0, keepdims=True)

    @pl.when(pl.program_id(0) == 0)
    def _():
        o_ref[...] = st

    @pl.when(pl.program_id(0) != 0)
    def _():
        o_ref[...] += st


def _bnvar(z, s):
    f = z.shape[1]
    return pl.pallas_call(
        _bnvar_body,
        grid=(GRID,),
        in_specs=[_row_spec(f), _full_spec((1, f))],
        out_specs=_full_spec((1, f)),
        out_shape=jax.ShapeDtypeStruct((1, f), jnp.float32),
    )(z, s)


def _bnapply_body(z_ref, sum_ref, var_ref, g_ref, b_ref, s_ref, o_ref, *,
                  mul_s):
    m = sum_ref[...] / N
    var = var_ref[...] / N
    sc = g_ref[...] * lax.rsqrt(var + EPS)
    sh = b_ref[...] - m * sc
    z = z_ref[...] * sc + sh
    if mul_s:
        z = z * s_ref[...]
    o_ref[...] = jnp.where(_rows(BLK) < N, z, 0.0)


def _bnapply(z, g, b, s, mul_s):
    f = z.shape[1]
    zsum = _bnsum(z)
    zvar = _bnvar(z, zsum)
    return pl.pallas_call(
        functools.partial(_bnapply_body, mul_s=mul_s),
        grid=(GRID,),
        in_specs=[_row_spec(f), _full_spec((1, f)), _full_spec((1, f)),
                  _full_spec((1, f)), _full_spec((1, f)), _row_spec(1)],
        out_specs=_row_spec(f),
        out_shape=jax.ShapeDtypeStruct((NP, f), jnp.float32),
    )(z, zsum, zvar, g.reshape(1, f), b.reshape(1, f), s)


def _gmax_body(gate_ref, bat_ref, o_ref):
    g = gate_ref[:, 0:1]
    onehot = bat_ref[...] == lax.broadcasted_iota(jnp.int32, (BLK, G), 1)
    valid = _rows(BLK) < N
    mg = jnp.where(onehot & valid, g, -jnp.inf)
    mx = jnp.max(mg, 0, keepdims=True)

    @pl.when(pl.program_id(0) == 0)
    def _():
        o_ref[...] = mx

    @pl.when(pl.program_id(0) != 0)
    def _():
        o_ref[...] = jnp.maximum(o_ref[...], mx)


def _gmax(gate, bat_col):
    return pl.pallas_call(
        _gmax_body,
        grid=(GRID,),
        in_specs=[_row_spec(128), _row_spec(1)],
        out_specs=_full_spec((1, G)),
        out_shape=jax.ShapeDtypeStruct((1, G), jnp.float32),
    )(gate, bat_col)


def _exp_weights(gate_ref, bat_ref, gmax_ref):
    g = gate_ref[:, 0:1]
    onehot = (bat_ref[...] ==
              lax.broadcasted_iota(jnp.int32, (BLK, G), 1)).astype(jnp.float32)
    gm = jnp.where(jnp.isfinite(gmax_ref[...]), gmax_ref[...], 0.0)
    gmrow = jnp.sum(onehot * gm, 1, keepdims=True)
    valid = (_rows(BLK) < N).astype(jnp.float32)
    e = jnp.exp(g - gmrow) * valid
    return onehot, e


def _denom_body(gate_ref, bat_ref, gmax_ref, o_ref):
    onehot, e = _exp_weights(gate_ref, bat_ref, gmax_ref)
    contrib = jnp.sum(onehot * e, 0, keepdims=True)

    @pl.when(pl.program_id(0) == 0)
    def _():
        o_ref[...] = contrib

    @pl.when(pl.program_id(0) != 0)
    def _():
        o_ref[...] += contrib


def _denom(gate, bat_col, gmax):
    return pl.pallas_call(
        _denom_body,
        grid=(GRID,),
        in_specs=[_row_spec(128), _row_spec(1), _full_spec((1, G))],
        out_specs=_full_spec((1, G)),
        out_shape=jax.ShapeDtypeStruct((1, G), jnp.float32),
    )(gate, bat_col, gmax)


def _pool_body(h_ref, gate_ref, bat_ref, batr_ref, gmax_ref, den_ref, o_ref):
    onehot, e = _exp_weights(gate_ref, bat_ref, gmax_ref)
    d = jnp.sum(onehot * den_ref[...], 1, keepdims=True)
    alpha = e / jnp.where(d > 0, d, 1.0)
    w = alpha * h_ref[...]
    onehot_t = (lax.broadcasted_iota(jnp.int32, (G, BLK), 0) ==
                batr_ref[...]).astype(jnp.float32)
    contrib = jnp.dot(onehot_t, w, preferred_element_type=jnp.float32, precision=lax.Precision.HIGHEST)

    @pl.when(pl.program_id(0) == 0)
    def _():
        o_ref[...] = contrib

    @pl.when(pl.program_id(0) != 0)
    def _():
        o_ref[...] += contrib


def _pool(h5, gate, bat_col, bat_row, gmax, den):
    return pl.pallas_call(
        _pool_body,
        grid=(GRID,),
        in_specs=[_row_spec(1024), _row_spec(128), _row_spec(1),
                  pl.BlockSpec((1, BLK), lambda i: (i * 0, i)),
                  _full_spec((1, G)), _full_spec((1, G))],
        out_specs=_full_spec((G, 1024)),
        out_shape=jax.ShapeDtypeStruct((G, 1024), jnp.float32),
    )(h5, gate, bat_col, bat_row, gmax, den)


def _head_body(p_ref, w2_ref, b2_ref, w3_ref, b3_ref, w4_ref, b4_ref, o_ref):
    z = jnp.dot(p_ref[...], w2_ref[...], preferred_element_type=jnp.float32)
    z = jnp.maximum(z + b2_ref[...], 0.0)
    z = jnp.dot(z, w3_ref[...], preferred_element_type=jnp.float32)
    z = jnp.maximum(z + b3_ref[...], 0.0)
    z = jnp.dot(z, w4_ref[...], preferred_element_type=jnp.float32)
    o_ref[...] = z + b4_ref[...]


def _head(p, w2, b2, w3p, b3p, w4p, b4p):
    return pl.pallas_call(
        _head_body,
        grid=(1,),
        in_specs=[_full_spec((G, 1024)), _full_spec((1024, 128)),
                  _full_spec((1, 128)), _full_spec((128, 128)),
                  _full_spec((1, 128)), _full_spec((128, 128)),
                  _full_spec((1, 128))],
        out_specs=_full_spec((G, 128)),
        out_shape=jax.ShapeDtypeStruct((G, 128), jnp.float32),
    )(p, w2, b2, w3p, b3p, w4p, b4p)


def kernel(x, edge_index, batch, params):
    x = x.astype(jnp.float32)
    row = edge_index[0].astype(jnp.int32)
    col = edge_index[1].astype(jnp.int32)
    bat = batch.astype(jnp.int32)

    pad_e = EP - E
    rows_g = jnp.concatenate(
        [row, jnp.zeros((pad_e,), jnp.int32)]).reshape(NBLK, 128)
    rows_s = jnp.concatenate(
        [row, jnp.full((pad_e,), DUMMY, jnp.int32)]).reshape(NBLK, 128)
    cols_s = jnp.concatenate(
        [col, jnp.full((pad_e,), DUMMY, jnp.int32)]).reshape(NBLK, 128)
    xp = jnp.pad(x, ((0, NP - N), (0, 128 - x.shape[1])))
    bat_col = jnp.pad(bat, (0, NP - N)).reshape(NP, 1)
    bat_row = bat_col.reshape(1, NP)
    ones128 = jnp.ones((128, 128), jnp.float32)
    zeros128 = jnp.zeros((ZR, 128), jnp.float32)

    w1p = jnp.pad(params['W1'], ((0, 128 - 29), (0, 0)))
    gwp = jnp.pad(params['gate_W'], ((0, 0), (0, 127)))
    gbp = jnp.pad(params['gate_b'], (0, 127)).reshape(1, 128)
    w3p = jnp.pad(params['fc3_W'], ((0, 0), (0, 112)))
    b3p = jnp.pad(params['fc3_b'], (0, 112)).reshape(1, 128)
    w4p = jnp.pad(params['fc4_W'], ((0, 112), (0, 127)))
    b4p = jnp.pad(params['fc4_b'], (0, 127)).reshape(1, 128)

    deg2 = _make_deg()(rows_s, ones128, zeros128)[:, :, 0]
    dinv = _dinv(deg2)

    def propagate(hp):
        f = hp.shape[1]
        nc = f // 128
        hpcs = [lax.slice_in_dim(hp, c * 128, (c + 1) * 128, axis=1)
                for c in range(nc)]
        return _make_scatter(nc)(*hpcs, rows_g, cols_s, zeros128)

    h = xp
    ws = [w1p, params['W2'], params['W3'], params['W4'], params['W5']]
    for i in range(1, 6):
        mp = _mm_scale(h, ws[i - 1], dinv)
        z = _combine(propagate(mp), mp, dinv, params['b%d' % i], True)
        h = _bnapply(z, params['bn%d_g' % i], params['bn%d_b' % i], dinv,
                     False)

    gate = _mm(h, gwp, gbp.reshape(128), act=False, mask=True)
    gmax = _gmax(gate, bat_col)
    den = _denom(gate, bat_col, gmax)
    pooled = _pool(h, gate, bat_col, bat_row, gmax, den)

    out = _head(pooled, params['fc2_W'],
                params['fc2_b'].reshape(1, 128), w3p, b3p, w4p, b4p)
    return out[:, :1]


# speedup vs baseline: 3.1824x; 1.3113x over previous
import functools

import jax
import jax.numpy as jnp
from jax import lax
from jax.experimental import pallas as pl
from jax.experimental.pallas import tpu as pltpu
from jax.experimental.pallas import tpu_sc as plsc

N = 10000
NP = 10240
E = 160000
EP = 163840
NBLK = EP // 128
BPW = NBLK // 32
ZR = NP // 16
G = 256
DUMMY = 10000
EPS = 1e-5
BLK = 1024
GRID = NP // BLK


def _sc_mesh():
    return plsc.VectorSubcoreMesh(core_axis_name="c", subcore_axis_name="s")


BPW2 = NBLK // 16


def _make_scatter(nc):
    K = 1

    def body(hp_all, rows, cols, zeros, out, ridx, cidx, gbuf, acc, gsem):
        cid = lax.axis_index("c")
        sid = lax.axis_index("s")
        pltpu.sync_copy(rows.at[pl.ds(sid * BPW2, BPW2)], ridx)
        pltpu.sync_copy(cols.at[pl.ds(sid * BPW2, BPW2)], cidx)

        for c in range(nc // 2):
            ch = jnp.int32(2 * c) + cid
            hp = hp_all.at[ch]
            pltpu.sync_copy(zeros, acc.at[pl.ds(sid * ZR, ZR)])
            plsc.subcore_barrier()

            def group(g, carry, hp=hp):
                base = g * jnp.int32(K)
                gathers = []
                for t in range(K):
                    gathers.append(pltpu.async_copy(
                        hp.at[ridx.at[base + jnp.int32(t)]],
                        gbuf.at[jnp.int32(t)], gsem))
                for t in range(K):
                    gathers[t].wait()
                    pltpu.sync_copy(gbuf.at[jnp.int32(t)],
                                    acc.at[cidx.at[base + jnp.int32(t)]],
                                    add=True)
                return carry

            lax.fori_loop(jnp.int32(0), jnp.int32(BPW2 // K), group, 0)
            plsc.subcore_barrier()
            pltpu.sync_copy(acc.at[pl.ds(sid * ZR, ZR)],
                            out.at[ch, pl.ds(sid * ZR, ZR)])

    return pl.kernel(
        body,
        out_type=jax.ShapeDtypeStruct((nc, NP, 128), jnp.float32),
        mesh=_sc_mesh(),
        scratch_types=[
            pltpu.VMEM((BPW2, 128), jnp.int32),
            pltpu.VMEM((BPW2, 128), jnp.int32),
            pltpu.VMEM((K, 128, 128), jnp.float32),
            pltpu.VMEM_SHARED((NP, 128), jnp.float32),
            pltpu.SemaphoreType.DMA,
        ],
    )


def _make_deg():
    def body(rows, ones, zeros, out, ridx, obuf, acc):
        cid = lax.axis_index("c")
        sid = lax.axis_index("s")
        wid = sid * 2 + cid
        pltpu.sync_copy(zeros, acc.at[pl.ds(sid * ZR, ZR)])
        plsc.subcore_barrier()
        pltpu.sync_copy(rows.at[pl.ds(wid * BPW, BPW)], ridx)
        pltpu.sync_copy(ones, obuf)

        def step(j, carry):
            pltpu.sync_copy(obuf, acc.at[ridx.at[j]], add=True)
            return carry

        lax.fori_loop(0, BPW, step, 0)
        plsc.subcore_barrier()
        pltpu.sync_copy(acc.at[pl.ds(sid * ZR, ZR)],
                        out.at[cid, pl.ds(sid * ZR, ZR)])

    return pl.kernel(
        body,
        out_type=jax.ShapeDtypeStruct((2, NP, 128), jnp.float32),
        mesh=_sc_mesh(),
        scratch_types=[
            pltpu.VMEM((BPW, 128), jnp.int32),
            pltpu.VMEM((128, 128), jnp.float32),
            pltpu.VMEM_SHARED((NP, 128), jnp.float32),
        ],
    )


def _rows(blk):
    return pl.program_id(0) * blk + lax.broadcasted_iota(jnp.int32, (blk, 1), 0)


def _row_spec(f, blk=BLK):
    return pl.BlockSpec((blk, f), lambda i: (i, i * 0))


def _full_spec(shape):
    nd = len(shape)
    return pl.BlockSpec(shape, lambda i: (i * 0,) * nd)


def _dinv_body(d_ref, o_ref):
    o_ref[...] = lax.rsqrt(d_ref[0] + d_ref[1] + 1.0)


def _dinv(deg2):
    return pl.pallas_call(
        _dinv_body,
        grid=(1,),
        in_specs=[_full_spec((2, 80, 128))],
        out_specs=_full_spec((80, 128)),
        out_shape=jax.ShapeDtypeStruct((80, 128), jnp.float32),
    )(deg2.reshape(2, 80, 128)).reshape(NP, 1)


def _scale_body(x_ref, s_ref, o_ref):
    o_ref[...] = x_ref[...] * s_ref[...]


def _scale(x, s):
    f = x.shape[1]
    return pl.pallas_call(
        _scale_body,
        grid=(GRID,),
        in_specs=[_row_spec(f), _row_spec(1)],
        out_specs=_row_spec(f),
        out_shape=jax.ShapeDtypeStruct((NP, f), jnp.float32),
    )(x, s)


def _mm_body(x_ref, w_ref, b_ref, o_ref, *, act, mask):
    z = jnp.dot(x_ref[...], w_ref[...], preferred_element_type=jnp.float32)
    z = z + b_ref[...]
    if act:
        z = jnp.maximum(z, 0.0)
    if mask:
        z = jnp.where(_rows(BLK) < N, z, 0.0)
    o_ref[...] = z


def _mm(x, w, b, act, mask):
    k, n = w.shape
    return pl.pallas_call(
        functools.partial(_mm_body, act=act, mask=mask),
        grid=(GRID,),
        in_specs=[_row_spec(k), _full_spec((k, n)), _full_spec((1, n))],
        out_specs=_row_spec(n),
        out_shape=jax.ShapeDtypeStruct((NP, n), jnp.float32),
    )(x, w, b.reshape(1, n))


def _mm_scale_body(x_ref, w_ref, s_ref, o_ref):
    z = jnp.dot(x_ref[...], w_ref[...], preferred_element_type=jnp.float32)
    o_ref[...] = z * s_ref[...]


def _mm_scale(x, w, s):
    k, n = w.shape
    return pl.pallas_call(
        _mm_scale_body,
        grid=(GRID,),
        in_specs=[_row_spec(k), _full_spec((k, n)), _row_spec(1)],
        out_specs=_row_spec(n),
        out_shape=jax.ShapeDtypeStruct((NP, n), jnp.float32),
    )(x, w, s)


def _combine_body(a_ref, hp_ref, s_ref, b_ref, o_ref, *, nc, bias_relu):
    parts = [a_ref[c] for c in range(nc)]
    acc = jnp.concatenate(parts, axis=1) if nc > 1 else parts[0]
    z = s_ref[...] * (acc + hp_ref[...])
    if bias_relu:
        z = jnp.maximum(z + b_ref[...], 0.0)
    o_ref[...] = jnp.where(_rows(BLK) < N, z, 0.0)


def _combine(acc_all, hp, s, b, bias_relu):
    f = hp.shape[1]
    nc = f // 128
    a_spec = pl.BlockSpec((nc, BLK, 128), lambda i: (i * 0, i, i * 0))
    return pl.pallas_call(
        functools.partial(_combine_body, nc=nc, bias_relu=bias_relu),
        grid=(GRID,),
        in_specs=[a_spec, _row_spec(f), _row_spec(1), _full_spec((1, f))],
        out_specs=_row_spec(f),
        out_shape=jax.ShapeDtypeStruct((NP, f), jnp.float32),
    )(acc_all, hp, s, b.reshape(1, f))


def _bnsum_body(z_ref, o_ref):
    st = jnp.sum(z_ref[...], 0, keepdims=True)

    @pl.when(pl.program_id(0) == 0)
    def _():
        o_ref[...] = st

    @pl.when(pl.program_id(0) != 0)
    def _():
        o_ref[...] += st


def _bnsum(z):
    f = z.shape[1]
    return pl.pallas_call(
        _bnsum_body,
        grid=(GRID,),
        in_specs=[_row_spec(f)],
        out_specs=_full_spec((1, f)),
        out_shape=jax.ShapeDtypeStruct((1, f), jnp.float32),
    )(z)


def _bnvar_body(z_ref, sum_ref, o_ref):
    d = z_ref[...] - sum_ref[...] / N
    d = jnp.where(_rows(BLK) < N, d, 0.0)
    st = jnp.sum(d * d, 0, keepdims=True)

    @pl.when(pl.program_id(0) == 0)
    def _():
        o_ref[...] = st

    @pl.when(pl.program_id(0) != 0)
    def _():
        o_ref[...] += st


def _bnvar(z, s):
    f = z.shape[1]
    return pl.pallas_call(
        _bnvar_body,
        grid=(GRID,),
        in_specs=[_row_spec(f), _full_spec((1, f))],
        out_specs=_full_spec((1, f)),
        out_shape=jax.ShapeDtypeStruct((1, f), jnp.float32),
    )(z, s)


def _bnapply_body(z_ref, sum_ref, var_ref, g_ref, b_ref, s_ref, o_ref, *,
                  mul_s):
    m = sum_ref[...] / N
    var = var_ref[...] / N
    sc = g_ref[...] * lax.rsqrt(var + EPS)
    sh = b_ref[...] - m * sc
    z = z_ref[...] * sc + sh
    if mul_s:
        z = z * s_ref[...]
    o_ref[...] = jnp.where(_rows(BLK) < N, z, 0.0)


def _bnapply(z, g, b, s, mul_s):
    f = z.shape[1]
    zsum = _bnsum(z)
    zvar = _bnvar(z, zsum)
    return pl.pallas_call(
        functools.partial(_bnapply_body, mul_s=mul_s),
        grid=(GRID,),
        in_specs=[_row_spec(f), _full_spec((1, f)), _full_spec((1, f)),
                  _full_spec((1, f)), _full_spec((1, f)), _row_spec(1)],
        out_specs=_row_spec(f),
        out_shape=jax.ShapeDtypeStruct((NP, f), jnp.float32),
    )(z, zsum, zvar, g.reshape(1, f), b.reshape(1, f), s)


def _gmax_body(gate_ref, bat_ref, o_ref):
    g = gate_ref[:, 0:1]
    onehot = bat_ref[...] == lax.broadcasted_iota(jnp.int32, (BLK, G), 1)
    valid = _rows(BLK) < N
    mg = jnp.where(onehot & valid, g, -jnp.inf)
    mx = jnp.max(mg, 0, keepdims=True)

    @pl.when(pl.program_id(0) == 0)
    def _():
        o_ref[...] = mx

    @pl.when(pl.program_id(0) != 0)
    def _():
        o_ref[...] = jnp.maximum(o_ref[...], mx)


def _gmax(gate, bat_col):
    return pl.pallas_call(
        _gmax_body,
        grid=(GRID,),
        in_specs=[_row_spec(128), _row_spec(1)],
        out_specs=_full_spec((1, G)),
        out_shape=jax.ShapeDtypeStruct((1, G), jnp.float32),
    )(gate, bat_col)


def _exp_weights(gate_ref, bat_ref, gmax_ref):
    g = gate_ref[:, 0:1]
    onehot = (bat_ref[...] ==
              lax.broadcasted_iota(jnp.int32, (BLK, G), 1)).astype(jnp.float32)
    gm = jnp.where(jnp.isfinite(gmax_ref[...]), gmax_ref[...], 0.0)
    gmrow = jnp.sum(onehot * gm, 1, keepdims=True)
    valid = (_rows(BLK) < N).astype(jnp.float32)
    e = jnp.exp(g - gmrow) * valid
    return onehot, e


def _denom_body(gate_ref, bat_ref, gmax_ref, o_ref):
    onehot, e = _exp_weights(gate_ref, bat_ref, gmax_ref)
    contrib = jnp.sum(onehot * e, 0, keepdims=True)

    @pl.when(pl.program_id(0) == 0)
    def _():
        o_ref[...] = contrib

    @pl.when(pl.program_id(0) != 0)
    def _():
        o_ref[...] += contrib


def _denom(gate, bat_col, gmax):
    return pl.pallas_call(
        _denom_body,
        grid=(GRID,),
        in_specs=[_row_spec(128), _row_spec(1), _full_spec((1, G))],
        out_specs=_full_spec((1, G)),
        out_shape=jax.ShapeDtypeStruct((1, G), jnp.float32),
    )(gate, bat_col, gmax)


def _pool_body(h_ref, gate_ref, bat_ref, batr_ref, gmax_ref, den_ref, o_ref):
    onehot, e = _exp_weights(gate_ref, bat_ref, gmax_ref)
    d = jnp.sum(onehot * den_ref[...], 1, keepdims=True)
    alpha = e / jnp.where(d > 0, d, 1.0)
    w = alpha * h_ref[...]
    onehot_t = (lax.broadcasted_iota(jnp.int32, (G, BLK), 0) ==
                batr_ref[...]).astype(jnp.float32)
    contrib = jnp.dot(onehot_t, w, preferred_element_type=jnp.float32, precision=lax.Precision.HIGHEST)

    @pl.when(pl.program_id(0) == 0)
    def _():
        o_ref[...] = contrib

    @pl.when(pl.program_id(0) != 0)
    def _():
        o_ref[...] += contrib


def _pool(h5, gate, bat_col, bat_row, gmax, den):
    return pl.pallas_call(
        _pool_body,
        grid=(GRID,),
        in_specs=[_row_spec(1024), _row_spec(128), _row_spec(1),
                  pl.BlockSpec((1, BLK), lambda i: (i * 0, i)),
                  _full_spec((1, G)), _full_spec((1, G))],
        out_specs=_full_spec((G, 1024)),
        out_shape=jax.ShapeDtypeStruct((G, 1024), jnp.float32),
    )(h5, gate, bat_col, bat_row, gmax, den)


def _head_body(p_ref, w2_ref, b2_ref, w3_ref, b3_ref, w4_ref, b4_ref, o_ref):
    z = jnp.dot(p_ref[...], w2_ref[...], preferred_element_type=jnp.float32)
    z = jnp.maximum(z + b2_ref[...], 0.0)
    z = jnp.dot(z, w3_ref[...], preferred_element_type=jnp.float32)
    z = jnp.maximum(z + b3_ref[...], 0.0)
    z = jnp.dot(z, w4_ref[...], preferred_element_type=jnp.float32)
    o_ref[...] = z + b4_ref[...]


def _head(p, w2, b2, w3p, b3p, w4p, b4p):
    return pl.pallas_call(
        _head_body,
        grid=(1,),
        in_specs=[_full_spec((G, 1024)), _full_spec((1024, 128)),
                  _full_spec((1, 128)), _full_spec((128, 128)),
                  _full_spec((1, 128)), _full_spec((128, 128)),
                  _full_spec((1, 128))],
        out_specs=_full_spec((G, 128)),
        out_shape=jax.ShapeDtypeStruct((G, 128), jnp.float32),
    )(p, w2, b2, w3p, b3p, w4p, b4p)


def kernel(x, edge_index, batch, params):
    x = x.astype(jnp.float32)
    row = edge_index[0].astype(jnp.int32)
    col = edge_index[1].astype(jnp.int32)
    bat = batch.astype(jnp.int32)

    pad_e = EP - E
    rows_g = jnp.concatenate(
        [row, jnp.zeros((pad_e,), jnp.int32)]).reshape(NBLK, 128)
    rows_s = jnp.concatenate(
        [row, jnp.full((pad_e,), DUMMY, jnp.int32)]).reshape(NBLK, 128)
    cols_s = jnp.concatenate(
        [col, jnp.full((pad_e,), DUMMY, jnp.int32)]).reshape(NBLK, 128)
    xp = jnp.pad(x, ((0, NP - N), (0, 128 - x.shape[1])))
    bat_col = jnp.pad(bat, (0, NP - N)).reshape(NP, 1)
    bat_row = bat_col.reshape(1, NP)
    ones128 = jnp.ones((128, 128), jnp.float32)
    zeros128 = jnp.zeros((ZR, 128), jnp.float32)

    w1p = jnp.pad(params['W1'], ((0, 128 - 29), (0, 0)))
    gwp = jnp.pad(params['gate_W'], ((0, 0), (0, 127)))
    gbp = jnp.pad(params['gate_b'], (0, 127)).reshape(1, 128)
    w3p = jnp.pad(params['fc3_W'], ((0, 0), (0, 112)))
    b3p = jnp.pad(params['fc3_b'], (0, 112)).reshape(1, 128)
    w4p = jnp.pad(params['fc4_W'], ((0, 112), (0, 127)))
    b4p = jnp.pad(params['fc4_b'], (0, 127)).reshape(1, 128)

    deg2 = _make_deg()(rows_s, ones128, zeros128)[:, :, 0]
    dinv = _dinv(deg2)

    def propagate(hp):
        f = hp.shape[1]
        nc = f // 128
        hp_all = jnp.transpose(hp.reshape(NP, nc, 128), (1, 0, 2))
        return _make_scatter(nc)(hp_all, rows_g, cols_s, zeros128)

    h = xp
    ws = [w1p, params['W2'], params['W3'], params['W4'], params['W5']]
    for i in range(1, 6):
        mp = _mm_scale(h, ws[i - 1], dinv)
        z = _combine(propagate(mp), mp, dinv, params['b%d' % i], True)
        h = _bnapply(z, params['bn%d_g' % i], params['bn%d_b' % i], dinv,
                     False)

    gate = _mm(h, gwp, gbp.reshape(128), act=False, mask=True)
    gmax = _gmax(gate, bat_col)
    den = _denom(gate, bat_col, gmax)
    pooled = _pool(h, gate, bat_col, bat_row, gmax, den)

    out = _head(pooled, params['fc2_W'],
                params['fc2_b'].reshape(1, 128), w3p, b3p, w4p, b4p)
    return out[:, :1]
